# baseline (device time: 128557 ns/iter reference)
import jax
import jax.numpy as jnp
from jax import lax
from jax.experimental import pallas as pl
from jax.experimental.pallas import tpu as pltpu

N_DEV = 4
B, SQ, SKV = 2, 512, 512
HQ_LOCAL, DH = 8, 64
D_MODEL = 768
D_HEADS = HQ_LOCAL * DH
BLK = 64


def kernel(x, Wq, K_ext, V_ext, Wo):
    my = lax.axis_index("i")
    K_loc = lax.dynamic_slice_in_dim(K_ext, my * HQ_LOCAL, HQ_LOCAL, axis=2)
    V_loc = lax.dynamic_slice_in_dim(V_ext, my * HQ_LOCAL, HQ_LOCAL, axis=2)
    K_loc = K_loc.transpose(0, 2, 1, 3).astype(jnp.bfloat16)
    V_loc = V_loc.transpose(0, 2, 1, 3).astype(jnp.bfloat16)
    x2 = x.reshape(B * SQ, D_MODEL).astype(jnp.bfloat16)
    Wq_b = Wq.astype(jnp.bfloat16)
    Wo_b = Wo.astype(jnp.bfloat16)

    def body(x_ref, wq_ref, k_ref, v_ref, wo_ref, out_ref,
             comm_ref, send_sems, recv_sems):
        my_pos = lax.axis_index("i")
        left = (my_pos - 1) % N_DEV
        right = (my_pos + 1) % N_DEV

        barrier_sem = pltpu.get_barrier_semaphore()
        for nbr in (left, right):
            pl.semaphore_signal(
                barrier_sem, inc=1,
                device_id=(nbr,), device_id_type=pl.DeviceIdType.MESH,
            )
        pl.semaphore_wait(barrier_sem, 2)

        q = jnp.dot(x_ref[...], wq_ref[...],
                    preferred_element_type=jnp.float32)

        qb = lax.broadcasted_iota(jnp.int32, (SQ, SKV), 0) // BLK
        kb = lax.broadcasted_iota(jnp.int32, (SQ, SKV), 1) // BLK
        mask = kb <= qb

        rows = []
        for b in range(B):
            cols = []
            for h in range(HQ_LOCAL):
                q_bh = q[b * SQ:(b + 1) * SQ, h * DH:(h + 1) * DH]
                q_bh = q_bh.astype(jnp.bfloat16)
                k_bh = k_ref[b, h]
                v_bh = v_ref[b, h]
                s = lax.dot_general(
                    q_bh, k_bh, (((1,), (1,)), ((), ())),
                    preferred_element_type=jnp.float32,
                ) * 0.125
                s = jnp.where(mask, s, -1e9)
                m = jnp.max(s, axis=-1, keepdims=True)
                w = jnp.exp(s - m)
                w = w / jnp.sum(w, axis=-1, keepdims=True)
                ctx_bh = jnp.dot(w.astype(jnp.bfloat16), v_bh,
                                 preferred_element_type=jnp.float32)
                cols.append(ctx_bh.astype(jnp.bfloat16))
            rows.append(jnp.concatenate(cols, axis=1))
        ctx = jnp.concatenate(rows, axis=0)

        partial = jnp.dot(ctx, wo_ref[...],
                          preferred_element_type=jnp.float32)

        comm_ref[0] = partial
        acc = partial
        for hop in range(N_DEV - 1):
            rdma = pltpu.make_async_remote_copy(
                src_ref=comm_ref.at[hop],
                dst_ref=comm_ref.at[hop + 1],
                send_sem=send_sems.at[hop],
                recv_sem=recv_sems.at[hop],
                device_id=(right,),
                device_id_type=pl.DeviceIdType.MESH,
            )
            rdma.start()
            rdma.wait()
            acc = acc + comm_ref[hop + 1]
        out_ref[...] = acc.reshape(B, SQ, D_MODEL)

    return pl.pallas_call(
        body,
        out_shape=jax.ShapeDtypeStruct((B, SQ, D_MODEL), jnp.float32),
        in_specs=[pl.BlockSpec(memory_space=pltpu.VMEM)] * 5,
        out_specs=pl.BlockSpec(memory_space=pltpu.VMEM),
        scratch_shapes=[
            pltpu.VMEM((N_DEV, B * SQ, D_MODEL), jnp.float32),
            pltpu.SemaphoreType.DMA((N_DEV - 1,)),
            pltpu.SemaphoreType.DMA((N_DEV - 1,)),
        ],
        compiler_params=pltpu.CompilerParams(collective_id=0),
    )(x2, Wq_b, K_loc, V_loc, Wo_b)


# device time: 48585 ns/iter; 2.6460x vs baseline; 2.6460x over previous
import jax
import jax.numpy as jnp
from jax import lax
from jax.experimental import pallas as pl
from jax.experimental.pallas import tpu as pltpu

N_DEV = 4
B, SQ, SKV = 2, 512, 512
HQ_LOCAL, DH = 8, 64
D_MODEL = 768
HALF = SQ
BLK = 64


def kernel(x, Wq, K_ext, V_ext, Wo):
    my = lax.axis_index("i")
    K_loc = lax.dynamic_slice_in_dim(K_ext, my * HQ_LOCAL, HQ_LOCAL, axis=2)
    V_loc = lax.dynamic_slice_in_dim(V_ext, my * HQ_LOCAL, HQ_LOCAL, axis=2)
    K_loc = K_loc.transpose(0, 2, 1, 3).astype(jnp.bfloat16)
    V_loc = V_loc.transpose(0, 2, 1, 3).astype(jnp.bfloat16)
    x2 = x.reshape(B * SQ, D_MODEL).astype(jnp.bfloat16)
    Wq_b = Wq.astype(jnp.bfloat16)
    Wo_b = Wo.astype(jnp.bfloat16)

    def body(x_ref, wq_ref, k_ref, v_ref, wo_ref, out_ref,
             sbufs, rbufs, send_sems, recv_sems):
        my_pos = lax.axis_index("i")
        p1 = jnp.bitwise_xor(my_pos, 1)
        p2 = 3 - my_pos
        keep_a = (my_pos == 0) | (my_pos == 3)
        first_b = jnp.where(keep_a, 1, 0)
        second_b = 1 - first_b

        barrier_sem = pltpu.get_barrier_semaphore()
        for nbr in (p1, p2):
            pl.semaphore_signal(
                barrier_sem, inc=1,
                device_id=(nbr,), device_id_type=pl.DeviceIdType.MESH,
            )
        pl.semaphore_wait(barrier_sem, 2)

        qb = lax.broadcasted_iota(jnp.int32, (SQ, SKV), 0) // BLK
        kb = lax.broadcasted_iota(jnp.int32, (SQ, SKV), 1) // BLK
        mask = kb <= qb

        def partial_for_batch(b):
            xb = x_ref[pl.ds(b * SQ, SQ), :]
            q = jnp.dot(xb, wq_ref[...],
                        preferred_element_type=jnp.float32)
            cols = []
            for h in range(HQ_LOCAL):
                q_bh = q[:, h * DH:(h + 1) * DH].astype(jnp.bfloat16)
                k_bh = k_ref[b, h]
                v_bh = v_ref[b, h]
                s = lax.dot_general(
                    q_bh, k_bh, (((1,), (1,)), ((), ())),
                    preferred_element_type=jnp.float32,
                ) * 0.125
                s = jnp.where(mask, s, -1e9)
                m = jnp.max(s, axis=-1, keepdims=True)
                w = jnp.exp(s - m)
                w = w / jnp.sum(w, axis=-1, keepdims=True)
                ctx_bh = jnp.dot(w.astype(jnp.bfloat16), v_bh,
                                 preferred_element_type=jnp.float32)
                cols.append(ctx_bh.astype(jnp.bfloat16))
            ctx = jnp.concatenate(cols, axis=1)
            return jnp.dot(ctx, wo_ref[...],
                           preferred_element_type=jnp.float32)

        def exchange(stage, peer):
            return pltpu.make_async_remote_copy(
                src_ref=sbufs.at[stage],
                dst_ref=rbufs.at[stage],
                send_sem=send_sems.at[stage],
                recv_sem=recv_sems.at[stage],
                device_id=(peer,),
                device_id_type=pl.DeviceIdType.MESH,
            )

        sbufs[0] = partial_for_batch(first_b).astype(jnp.bfloat16)
        rdma1 = exchange(0, p1)
        rdma1.start()
        mine = partial_for_batch(second_b)
        rdma1.wait()
        acc = mine + rbufs[0].astype(jnp.float32)

        sbufs[1] = acc.astype(jnp.bfloat16)
        rdma2 = exchange(1, p2)
        rdma2.start()
        rdma2.wait()
        red = acc + rbufs[1].astype(jnp.float32)

        sbufs[2] = red.astype(jnp.bfloat16)
        rdma3 = exchange(2, p1)
        rdma3.start()
        rdma3.wait()

        keep_off = second_b * SQ
        out_ref[pl.ds(keep_off, HALF), :] = red
        out_ref[pl.ds(first_b * SQ, HALF), :] = rbufs[2].astype(jnp.float32)

    out = pl.pallas_call(
        body,
        out_shape=jax.ShapeDtypeStruct((B * SQ, D_MODEL), jnp.float32),
        in_specs=[pl.BlockSpec(memory_space=pltpu.VMEM)] * 5,
        out_specs=pl.BlockSpec(memory_space=pltpu.VMEM),
        scratch_shapes=[
            pltpu.VMEM((3, HALF, D_MODEL), jnp.bfloat16),
            pltpu.VMEM((3, HALF, D_MODEL), jnp.bfloat16),
            pltpu.SemaphoreType.DMA((3,)),
            pltpu.SemaphoreType.DMA((3,)),
        ],
        compiler_params=pltpu.CompilerParams(collective_id=0),
    )(x2, Wq_b, K_loc, V_loc, Wo_b)
    return out.reshape(B, SQ, D_MODEL)


# device time: 34115 ns/iter; 3.7683x vs baseline; 1.4242x over previous
import jax
import jax.numpy as jnp
from jax import lax
from jax.experimental import pallas as pl
from jax.experimental.pallas import tpu as pltpu

N_DEV = 4
B, SQ, SKV = 2, 512, 512
HQ_LOCAL, DH = 8, 64
D_MODEL = 768
HALF = SQ
NC = 4
CH = HALF // NC
BLK = 64


def kernel(x, Wq, K_ext, V_ext, Wo):
    my = lax.axis_index("i")
    K_loc = lax.dynamic_slice_in_dim(K_ext, my * HQ_LOCAL, HQ_LOCAL, axis=2)
    V_loc = lax.dynamic_slice_in_dim(V_ext, my * HQ_LOCAL, HQ_LOCAL, axis=2)
    K_loc = K_loc.astype(jnp.bfloat16).transpose(0, 2, 1, 3)
    V_loc = V_loc.astype(jnp.bfloat16).transpose(0, 2, 1, 3)
    x2 = x.reshape(B * SQ, D_MODEL)

    def body(x_ref, wq_ref, k_ref, v_ref, wo_ref, out_ref,
             sbufs, rbufs, send_sems, recv_sems):
        my_pos = lax.axis_index("i")
        p1 = jnp.bitwise_xor(my_pos, 1)
        p2 = 3 - my_pos
        keep_a = (my_pos == 0) | (my_pos == 3)
        first_b = jnp.where(keep_a, 1, 0)
        second_b = 1 - first_b

        barrier_sem = pltpu.get_barrier_semaphore()
        for nbr in (p1, p2):
            pl.semaphore_signal(
                barrier_sem, inc=1,
                device_id=(nbr,), device_id_type=pl.DeviceIdType.MESH,
            )
        pl.semaphore_wait(barrier_sem, 2)

        wq = wq_ref[...].astype(jnp.bfloat16)
        wo = wo_ref[...].astype(jnp.bfloat16)
        qb = lax.broadcasted_iota(jnp.int32, (SQ, SKV), 0) // BLK
        kb = lax.broadcasted_iota(jnp.int32, (SQ, SKV), 1) // BLK
        maskf = (kb <= qb).astype(jnp.float32)

        def partial_for_batch(b):
            xb = x_ref[pl.ds(b * SQ, SQ), :].astype(jnp.bfloat16)
            q = jnp.dot(xb, wq, preferred_element_type=jnp.float32)
            cols = []
            for h in range(HQ_LOCAL):
                q_bh = (q[:, h * DH:(h + 1) * DH] * 0.125).astype(jnp.bfloat16)
                k_bh = k_ref[b, h]
                v_bh = v_ref[b, h]
                s = lax.dot_general(
                    q_bh, k_bh, (((1,), (1,)), ((), ())),
                    preferred_element_type=jnp.float32,
                )
                w = jnp.exp(s) * maskf
                denom = jnp.sum(w, axis=-1, keepdims=True)
                ctx_bh = jnp.dot(w.astype(jnp.bfloat16), v_bh,
                                 preferred_element_type=jnp.float32)
                cols.append((ctx_bh / denom).astype(jnp.bfloat16))
            ctx = jnp.concatenate(cols, axis=1)
            return jnp.dot(ctx, wo,
                           preferred_element_type=jnp.float32)

        def exchange(stage, c, peer):
            rows = pl.ds(c * CH, CH)
            return pltpu.make_async_remote_copy(
                src_ref=sbufs.at[stage, rows, :],
                dst_ref=rbufs.at[stage, rows, :],
                send_sem=send_sems.at[stage, c],
                recv_sem=recv_sems.at[stage, c],
                device_id=(peer,),
                device_id_type=pl.DeviceIdType.MESH,
            )

        sbufs[0] = partial_for_batch(first_b).astype(jnp.bfloat16)
        r1 = [exchange(0, c, p1) for c in range(NC)]
        for r in r1:
            r.start()
        mine = partial_for_batch(second_b)

        acc_cs, r2 = [], []
        for c in range(NC):
            r1[c].wait()
            lo, hi = c * CH, (c + 1) * CH
            acc_c = mine[lo:hi, :] + rbufs[0, lo:hi, :].astype(jnp.float32)
            sbufs[1, lo:hi, :] = acc_c.astype(jnp.bfloat16)
            r = exchange(1, c, p2)
            r.start()
            r2.append(r)
            acc_cs.append(acc_c)

        red_cs, r3 = [], []
        for c in range(NC):
            r2[c].wait()
            lo, hi = c * CH, (c + 1) * CH
            red_c = acc_cs[c] + rbufs[1, lo:hi, :].astype(jnp.float32)
            sbufs[2, lo:hi, :] = red_c.astype(jnp.bfloat16)
            r = exchange(2, c, p1)
            r.start()
            r3.append(r)
            red_cs.append(red_c)

        keep_off = second_b * SQ
        for c in range(NC):
            out_ref[pl.ds(keep_off + c * CH, CH), :] = red_cs[c]
        for r in r3:
            r.wait()
        out_ref[pl.ds(first_b * SQ, HALF), :] = rbufs[2].astype(jnp.float32)

    out = pl.pallas_call(
        body,
        out_shape=jax.ShapeDtypeStruct((B * SQ, D_MODEL), jnp.float32),
        in_specs=[pl.BlockSpec(memory_space=pltpu.VMEM)] * 5,
        out_specs=pl.BlockSpec(memory_space=pltpu.VMEM),
        scratch_shapes=[
            pltpu.VMEM((3, HALF, D_MODEL), jnp.bfloat16),
            pltpu.VMEM((3, HALF, D_MODEL), jnp.bfloat16),
            pltpu.SemaphoreType.DMA((3, NC)),
            pltpu.SemaphoreType.DMA((3, NC)),
        ],
        compiler_params=pltpu.CompilerParams(collective_id=0),
    )(x2, Wq, K_loc, V_loc, Wo)
    return out.reshape(B, SQ, D_MODEL)


# device time: 30159 ns/iter; 4.2626x vs baseline; 1.1312x over previous
import jax
import jax.numpy as jnp
from jax import lax
from jax.experimental import pallas as pl
from jax.experimental.pallas import tpu as pltpu

N_DEV = 4
B, SQ, SKV = 2, 512, 512
HQ_LOCAL, DH = 8, 64
D_MODEL = 768
HALF = SQ
NSLAB = 2
SLAB = HALF // NSLAB
NC = 4
CH = HALF // NC
CPS = NC // NSLAB
BLK = 64


def kernel(x, Wq, K_ext, V_ext, Wo):
    my = lax.axis_index("i")
    K_loc = lax.dynamic_slice_in_dim(K_ext, my * HQ_LOCAL, HQ_LOCAL, axis=2)
    V_loc = lax.dynamic_slice_in_dim(V_ext, my * HQ_LOCAL, HQ_LOCAL, axis=2)
    K_loc = K_loc.astype(jnp.bfloat16).transpose(0, 2, 1, 3)
    V_loc = V_loc.astype(jnp.bfloat16).transpose(0, 2, 1, 3)
    x2 = x.reshape(B * SQ, D_MODEL)

    def body(x_ref, wq_ref, k_ref, v_ref, wo_ref, out_ref,
             sbufs, rbufs, send_sems, recv_sems):
        my_pos = lax.axis_index("i")
        p1 = jnp.bitwise_xor(my_pos, 1)
        p2 = 3 - my_pos
        keep_a = (my_pos == 0) | (my_pos == 3)
        first_b = jnp.where(keep_a, 1, 0)
        second_b = 1 - first_b

        barrier_sem = pltpu.get_barrier_semaphore()
        for nbr in (p1, p2):
            pl.semaphore_signal(
                barrier_sem, inc=1,
                device_id=(nbr,), device_id_type=pl.DeviceIdType.MESH,
            )
        pl.semaphore_wait(barrier_sem, 2)

        wq = wq_ref[...].astype(jnp.bfloat16)
        wo = wo_ref[...].astype(jnp.bfloat16)
        qb = lax.broadcasted_iota(jnp.int32, (SQ, SKV), 0) // BLK
        kb = lax.broadcasted_iota(jnp.int32, (SQ, SKV), 1) // BLK
        maskf = (kb <= qb).astype(jnp.float32)

        def partial_slab(b, s):
            kvlen = (s + 1) * SLAB
            rows = pl.ds(b * SQ + s * SLAB, SLAB)
            xb = x_ref[rows, :].astype(jnp.bfloat16)
            q = jnp.dot(xb, wq, preferred_element_type=jnp.float32)
            mask_s = maskf[s * SLAB:(s + 1) * SLAB, :kvlen]
            cols = []
            for h in range(HQ_LOCAL):
                q_bh = (q[:, h * DH:(h + 1) * DH] * 0.125).astype(jnp.bfloat16)
                k_bh = k_ref[b, h][:kvlen, :]
                v_bh = v_ref[b, h][:kvlen, :]
                sc = lax.dot_general(
                    q_bh, k_bh, (((1,), (1,)), ((), ())),
                    preferred_element_type=jnp.float32,
                )
                w = jnp.exp(sc) * mask_s
                denom = jnp.sum(w, axis=-1, keepdims=True)
                ctx_bh = jnp.dot(w.astype(jnp.bfloat16), v_bh,
                                 preferred_element_type=jnp.float32)
                cols.append((ctx_bh / denom).astype(jnp.bfloat16))
            ctx = jnp.concatenate(cols, axis=1)
            return jnp.dot(ctx, wo,
                           preferred_element_type=jnp.float32)

        def exchange(stage, c, peer):
            rows = pl.ds(c * CH, CH)
            return pltpu.make_async_remote_copy(
                src_ref=sbufs.at[stage, rows, :],
                dst_ref=rbufs.at[stage, rows, :],
                send_sem=send_sems.at[stage, c],
                recv_sem=recv_sems.at[stage, c],
                device_id=(peer,),
                device_id_type=pl.DeviceIdType.MESH,
            )

        r1 = [exchange(0, c, p1) for c in range(NC)]
        for s in range(NSLAB):
            p = partial_slab(first_b, s)
            sbufs[0, s * SLAB:(s + 1) * SLAB, :] = p.astype(jnp.bfloat16)
            for j in range(CPS):
                r1[s * CPS + j].start()

        acc_cs, r2 = [], []
        for s in range(NSLAB):
            mine_s = partial_slab(second_b, s)
            for j in range(CPS):
                c = s * CPS + j
                lo, hi = c * CH, (c + 1) * CH
                jlo, jhi = j * CH, (j + 1) * CH
                r1[c].wait()
                acc_c = mine_s[jlo:jhi, :] + rbufs[0, lo:hi, :].astype(
                    jnp.float32)
                sbufs[1, lo:hi, :] = acc_c.astype(jnp.bfloat16)
                r = exchange(1, c, p2)
                r.start()
                r2.append(r)
                acc_cs.append(acc_c)

        red_cs, r3 = [], []
        for c in range(NC):
            r2[c].wait()
            lo, hi = c * CH, (c + 1) * CH
            red_c = acc_cs[c] + rbufs[1, lo:hi, :].astype(jnp.float32)
            sbufs[2, lo:hi, :] = red_c.astype(jnp.bfloat16)
            r = exchange(2, c, p1)
            r.start()
            r3.append(r)
            red_cs.append(red_c)

        keep_off = second_b * SQ
        for c in range(NC):
            out_ref[pl.ds(keep_off + c * CH, CH), :] = red_cs[c].astype(
                jnp.bfloat16)
        for r in r3:
            r.wait()
        out_ref[pl.ds(first_b * SQ, HALF), :] = rbufs[2]

    out = pl.pallas_call(
        body,
        out_shape=jax.ShapeDtypeStruct((B * SQ, D_MODEL), jnp.bfloat16),
        in_specs=[pl.BlockSpec(memory_space=pltpu.VMEM)] * 5,
        out_specs=pl.BlockSpec(memory_space=pltpu.VMEM),
        scratch_shapes=[
            pltpu.VMEM((3, HALF, D_MODEL), jnp.bfloat16),
            pltpu.VMEM((3, HALF, D_MODEL), jnp.bfloat16),
            pltpu.SemaphoreType.DMA((3, NC)),
            pltpu.SemaphoreType.DMA((3, NC)),
        ],
        compiler_params=pltpu.CompilerParams(collective_id=0),
    )(x2, Wq, K_loc, V_loc, Wo)
    return out.reshape(B, SQ, D_MODEL)
